# baseline (speedup 1.0000x reference)
import functools

import jax
import jax.numpy as jnp
from jax import lax
from jax.experimental import pallas as pl
from jax.experimental.pallas import tpu as pltpu
from jax.experimental.pallas import tpu_sc as plsc

_D = 64
_CHUNK = 128
_NC = 2
_NS = 16
_NW = _NC * _NS


def _rsqrt16(v):
    i = plsc.bitcast(v, jnp.int32)
    i = jnp.int32(0x5F3759DF) - lax.shift_right_logical(i, 1)
    y = plsc.bitcast(i, jnp.float32)
    y = y * (1.5 - (0.5 * v) * y * y)
    return y


def _lane_sum(v, perms):
    for p in perms:
        v = v + jnp.take_along_axis(v, p, axis=0)
    return v


def _build(nchunks, seq_len):
    mesh = plsc.VectorSubcoreMesh(core_axis_name="c", subcore_axis_name="s")

    @functools.partial(
        pl.kernel,
        mesh=mesh,
        compiler_params=pltpu.CompilerParams(
            needs_layout_passes=False, use_tc_tiling_on_sc=False),
        out_type=jax.ShapeDtypeStruct((_NW, nchunks, _CHUNK, _D), jnp.float32),
        scratch_types=[
            pltpu.VMEM((nchunks, _CHUNK), jnp.int32),
            pltpu.VMEM((_D,), jnp.float32),
            pltpu.VMEM((_D,), jnp.float32),
            pltpu.VMEM((_CHUNK, _D), jnp.float32),
            pltpu.VMEM((_CHUNK, _D), jnp.float32),
            pltpu.VMEM((_CHUNK, _D), jnp.float32),
            pltpu.VMEM((_CHUNK, _D), jnp.float32),
            pltpu.SemaphoreType.DMA,
            pltpu.SemaphoreType.DMA,
            pltpu.SemaphoreType.DMA,
            pltpu.SemaphoreType.DMA,
            pltpu.SemaphoreType.DMA,
            pltpu.SemaphoreType.DMA,
        ],
    )
    def k(idx_hbm, table_hbm, pos2_hbm, g_hbm, b_hbm, out_hbm,
          idx_v, g_v, b_v, ibuf0, ibuf1, obuf0, obuf1,
          gsem0, gsem1, ssem0, ssem1, psem0, psem1):
        wid = lax.axis_index("s") * _NC + lax.axis_index("c")
        pltpu.sync_copy(idx_hbm.at[wid], idx_v)
        pltpu.sync_copy(g_hbm, g_v)
        pltpu.sync_copy(b_hbm, b_v)
        gk = [g_v[pl.ds(t * 16, 16)] for t in range(4)]
        bk = [b_v[pl.ds(t * 16, 16)] for t in range(4)]
        lanes = lax.iota(jnp.int32, 16)
        perms = [lanes ^ (1 << e) for e in range(4)]

        ibufs = (ibuf0, ibuf1)
        obufs = (obuf0, obuf1)
        gsems = (gsem0, gsem1)
        ssems = (ssem0, ssem1)
        psems = (psem0, psem1)

        def pos_src(j):
            pb = lax.rem(j * _CHUNK, seq_len)
            return pos2_hbm.at[pl.ds(pb, _CHUNK)]

        def start_prefill(j, b):
            pltpu.async_copy(pos_src(j), ibufs[b], psems[b])

        def issue_gather(j, b):
            pltpu.make_async_copy(pos_src(j), ibufs[b], psems[b]).wait()
            pltpu.async_copy(
                table_hbm.at[idx_v.at[j]], ibufs[b], gsems[b], add=True)

        def wait_gather(j, b):
            pltpu.make_async_copy(
                table_hbm.at[idx_v.at[j]], ibufs[b], gsems[b]).wait()

        def wait_scatter(j, b):
            pltpu.make_async_copy(
                obufs[b], out_hbm.at[wid, j], ssems[b]).wait()

        start_prefill(0, 0)
        issue_gather(0, 0)

        def do_chunk(j, b):
            @pl.when(j + 1 < nchunks)
            def _():
                start_prefill(j + 1, 1 - b)
            wait_gather(j, b)

            @pl.when(j + 1 < nchunks)
            def _():
                issue_gather(j + 1, 1 - b)

            @pl.when(j >= 2)
            def _():
                wait_scatter(j - 2, b)

            ibuf = ibufs[b]
            obuf = obufs[b]

            @plsc.parallel_loop(0, _CHUNK, 1, unroll=8)
            def row(i):
                y = [ibuf[i, pl.ds(t * 16, 16)] for t in range(4)]
                s = (y[0] + y[1]) + (y[2] + y[3])
                q = (y[0] * y[0] + y[1] * y[1]) + (y[2] * y[2] + y[3] * y[3])
                mean = _lane_sum(s, perms) * (1.0 / _D)
                msq = _lane_sum(q, perms) * (1.0 / _D)
                var = msq - mean * mean + 1e-5
                r = _rsqrt16(var)
                for t in range(4):
                    rg = r * gk[t]
                    obuf[i, pl.ds(t * 16, 16)] = y[t] * rg + (bk[t] - mean * rg)

            pltpu.async_copy(obuf, out_hbm.at[wid, j], ssems[b])

        def outer(t, c):
            do_chunk(2 * t, 0)
            do_chunk(2 * t + 1, 1)
            return c

        lax.fori_loop(0, nchunks // 2, outer, 0)
        wait_scatter(nchunks - 2, 0)
        wait_scatter(nchunks - 1, 1)

    return k


def kernel(token_ids, token_table, pos_table, gamma, beta):
    B, L = token_ids.shape
    V, D = token_table.shape
    assert D == _D and pos_table.shape == (L, D)
    total = B * L
    assert total % (_NW * _CHUNK) == 0
    nchunks = total // (_NW * _CHUNK)
    idx3 = token_ids.astype(jnp.int32).reshape(_NW, nchunks, _CHUNK)
    pos2 = jnp.concatenate([pos_table, pos_table], axis=0).astype(jnp.float32)
    out = _build(nchunks, L)(
        idx3,
        token_table.astype(jnp.float32),
        pos2,
        gamma.astype(jnp.float32),
        beta.astype(jnp.float32),
    )
    return out.reshape(B, L, D)


# speedup vs baseline: 1.1835x; 1.1835x over previous
import functools

import jax
import jax.numpy as jnp
from jax import lax
from jax.experimental import pallas as pl
from jax.experimental.pallas import tpu as pltpu
from jax.experimental.pallas import tpu_sc as plsc

_D = 64
_DP = 128
_CHUNK = 128
_NC = 2
_NS = 16
_NW = _NC * _NS


def _rsqrt16(v):
    i = plsc.bitcast(v, jnp.int32)
    i = jnp.int32(0x5F3759DF) - lax.shift_right_logical(i, 1)
    y = plsc.bitcast(i, jnp.float32)
    y = y * (1.5 - (0.5 * v) * y * y)
    y = y * (1.5 - (0.5 * v) * y * y)
    return y


def _build(nchunks, seq_len, nrows):
    mesh = plsc.VectorSubcoreMesh(core_axis_name="c", subcore_axis_name="s")

    @functools.partial(
        pl.kernel,
        mesh=mesh,
        compiler_params=pltpu.CompilerParams(
            needs_layout_passes=False, use_tc_tiling_on_sc=False),
        out_type=jax.ShapeDtypeStruct((nrows, _DP), jnp.float32),
        scratch_types=[
            pltpu.VMEM((nchunks, _CHUNK), jnp.int32),
            pltpu.VMEM((_D,), jnp.float32),
            pltpu.VMEM((_D,), jnp.float32),
            pltpu.VMEM((_CHUNK, _DP), jnp.float32),
            pltpu.VMEM((_CHUNK, _DP), jnp.float32),
            pltpu.VMEM((_CHUNK, _DP), jnp.float32),
            pltpu.VMEM((_CHUNK, _DP), jnp.float32),
            pltpu.SemaphoreType.DMA,
            pltpu.SemaphoreType.DMA,
            pltpu.SemaphoreType.DMA,
            pltpu.SemaphoreType.DMA,
            pltpu.SemaphoreType.DMA,
            pltpu.SemaphoreType.DMA,
        ],
    )
    def k(idx_hbm, table_hbm, pos2_hbm, g_hbm, b_hbm, out_hbm,
          idx_v, g_v, b_v, ibuf0, ibuf1, obuf0, obuf1,
          gsem0, gsem1, ssem0, ssem1, psem0, psem1):
        wid = lax.axis_index("s") * _NC + lax.axis_index("c")
        row0 = wid * (nchunks * _CHUNK)
        pltpu.sync_copy(idx_hbm.at[wid], idx_v)
        pltpu.sync_copy(g_hbm, g_v)
        pltpu.sync_copy(b_hbm, b_v)
        gk = [g_v[pl.ds(t * 16, 16)] for t in range(4)]
        bk = [b_v[pl.ds(t * 16, 16)] for t in range(4)]

        ibufs = (ibuf0, ibuf1)
        obufs = (obuf0, obuf1)
        gsems = (gsem0, gsem1)
        ssems = (ssem0, ssem1)
        psems = (psem0, psem1)

        def pos_src(j):
            pb = lax.rem(j * _CHUNK, seq_len)
            return pos2_hbm.at[pl.ds(pb, _CHUNK)]

        def start_prefill(j, b):
            pltpu.async_copy(pos_src(j), ibufs[b], psems[b])

        def issue_gather(j, b):
            pltpu.make_async_copy(pos_src(j), ibufs[b], psems[b]).wait()
            pltpu.async_copy(
                table_hbm.at[idx_v.at[j]], ibufs[b], gsems[b], add=True)

        def wait_gather(j, b):
            pltpu.make_async_copy(
                table_hbm.at[idx_v.at[j]], ibufs[b], gsems[b]).wait()

        def out_dst(j):
            return out_hbm.at[pl.ds(row0 + j * _CHUNK, _CHUNK)]

        def wait_scatter(j, b):
            pltpu.make_async_copy(obufs[b], out_dst(j), ssems[b]).wait()

        start_prefill(0, 0)
        issue_gather(0, 0)

        def do_chunk(j, b):
            @pl.when(j + 1 < nchunks)
            def _():
                start_prefill(j + 1, 1 - b)
            wait_gather(j, b)

            @pl.when(j + 1 < nchunks)
            def _():
                issue_gather(j + 1, 1 - b)

            @pl.when(j >= 2)
            def _():
                wait_scatter(j - 2, b)

            ibuf = ibufs[b]
            obuf = obufs[b]

            @plsc.parallel_loop(0, _CHUNK, 1, unroll=8)
            def row(i):
                y = [ibuf[i, pl.ds(t * 16, 16)] for t in range(4)]
                s = (y[0] + y[1]) + (y[2] + y[3])
                q = (y[0] * y[0] + y[1] * y[1]) + (y[2] * y[2] + y[3] * y[3])
                mean = jnp.sum(s) * (1.0 / _D)
                msq = jnp.sum(q) * (1.0 / _D)
                var = msq - mean * mean + 1e-5
                r = _rsqrt16(jnp.broadcast_to(var, (16,)))
                for t in range(4):
                    rg = r * gk[t]
                    obuf[i, pl.ds(t * 16, 16)] = y[t] * rg + (bk[t] - mean * rg)

            pltpu.async_copy(obuf, out_dst(j), ssems[b])

        def outer(t, c):
            do_chunk(2 * t, 0)
            do_chunk(2 * t + 1, 1)
            return c

        lax.fori_loop(0, nchunks // 2, outer, 0)
        wait_scatter(nchunks - 2, 0)
        wait_scatter(nchunks - 1, 1)

    return k


def kernel(token_ids, token_table, pos_table, gamma, beta):
    B, L = token_ids.shape
    V, D = token_table.shape
    assert D == _D and pos_table.shape == (L, D)
    total = B * L
    assert total % (_NW * _CHUNK) == 0
    nchunks = total // (_NW * _CHUNK)
    idx3 = token_ids.astype(jnp.int32).reshape(_NW, nchunks, _CHUNK)
    table_p = jnp.pad(token_table.astype(jnp.float32), ((0, 0), (0, _DP - _D)))
    pos2 = jnp.concatenate([pos_table, pos_table], axis=0).astype(jnp.float32)
    pos2_p = jnp.pad(pos2, ((0, 0), (0, _DP - _D)))
    out = _build(nchunks, L, total)(
        idx3,
        table_p,
        pos2_p,
        gamma.astype(jnp.float32),
        beta.astype(jnp.float32),
    )
    return out[:, :_D].reshape(B, L, D)


# speedup vs baseline: 1.3088x; 1.1058x over previous
import functools

import jax
import jax.numpy as jnp
from jax import lax
from jax.experimental import pallas as pl
from jax.experimental.pallas import tpu as pltpu
from jax.experimental.pallas import tpu_sc as plsc

_D = 64
_DP = 128
_CHUNK = 128
_NC = 2
_NS = 16
_NW = _NC * _NS
_NSEM = 8


def _build_gather(nchunks, nrows):
    mesh = plsc.VectorSubcoreMesh(core_axis_name="c", subcore_axis_name="s")

    @functools.partial(
        pl.kernel,
        mesh=mesh,
        compiler_params=pltpu.CompilerParams(
            needs_layout_passes=False, use_tc_tiling_on_sc=False),
        out_type=jax.ShapeDtypeStruct((nrows, _DP), jnp.float32),
        scratch_types=[
            pltpu.VMEM((nchunks, _CHUNK), jnp.int32),
            pltpu.VMEM((_CHUNK, _D), jnp.float32),
            pltpu.VMEM((_CHUNK, _D), jnp.float32),
            pltpu.SemaphoreType.DMA,
            pltpu.SemaphoreType.DMA,
            pltpu.SemaphoreType.DMA,
            pltpu.SemaphoreType.DMA,
        ],
    )
    def k(idx_hbm, table_hbm, out_hbm, idx_v, ibuf0, ibuf1,
          gsem0, gsem1, ssem0, ssem1):
        wid = lax.axis_index("s") * _NC + lax.axis_index("c")
        row0 = wid * (nchunks * _CHUNK)
        pltpu.sync_copy(idx_hbm.at[wid], idx_v)
        ibufs = (ibuf0, ibuf1)
        gsems = (gsem0, gsem1)
        ssems = (ssem0, ssem1)

        def dst(j):
            return out_hbm.at[pl.ds(row0 + j * _CHUNK, _CHUNK), pl.ds(0, _D)]

        def issue_gather(j, b):
            pltpu.async_copy(table_hbm.at[idx_v.at[j]], ibufs[b], gsems[b])

        def wait_gather(j, b):
            pltpu.make_async_copy(
                table_hbm.at[idx_v.at[j]], ibufs[b], gsems[b]).wait()

        def issue_scatter(j, b):
            pltpu.async_copy(ibufs[b], dst(j), ssems[b])

        def wait_scatter(j, b):
            pltpu.make_async_copy(ibufs[b], dst(j), ssems[b]).wait()

        issue_gather(0, 0)

        def body(t, c):
            for u in range(2):
                j = 2 * t + u
                wait_gather(j, u)

                @pl.when(j + 1 < nchunks)
                def _():
                    @pl.when(j >= 1)
                    def _():
                        wait_scatter(j - 1, 1 - u)

                    issue_gather(j + 1, 1 - u)

                issue_scatter(j, u)
            return c

        lax.fori_loop(0, nchunks // 2, body, 0)
        wait_scatter(nchunks - 2, 0)
        wait_scatter(nchunks - 1, 1)

    return k


_RB = 3200


def _ln_block(x_ref, pos_ref, g_ref, b_ref, o_ref):
    x = x_ref[...][:, :_D] + pos_ref[...]
    m = jnp.mean(x, axis=1, keepdims=True)
    c = x - m
    v = jnp.mean(c * c, axis=1, keepdims=True)
    y = c * lax.rsqrt(v + 1e-5) * g_ref[...] + b_ref[...]
    o_ref[...] = y.reshape(_RB // 200, 200, _D)


def _ln_apply(x2, pos_t, gamma, beta, batch, seq_len):
    nrows = x2.shape[0]
    grid = nrows // _RB
    return pl.pallas_call(
        _ln_block,
        grid=(grid,),
        in_specs=[
            pl.BlockSpec((_RB, _DP), lambda i: (i, 0)),
            pl.BlockSpec((_RB, _D), lambda i: (0, 0)),
            pl.BlockSpec((1, _D), lambda i: (0, 0)),
            pl.BlockSpec((1, _D), lambda i: (0, 0)),
        ],
        out_specs=pl.BlockSpec(
            (_RB // 200, 200, _D), lambda i: (i, 0, 0)),
        out_shape=jax.ShapeDtypeStruct((batch, seq_len, _D), jnp.float32),
    )(x2, pos_t, gamma, beta)


def kernel(token_ids, token_table, pos_table, gamma, beta):
    B, L = token_ids.shape
    V, D = token_table.shape
    assert D == _D and pos_table.shape == (L, D)
    total = B * L
    assert total % (_NW * _CHUNK) == 0
    nchunks = total // (_NW * _CHUNK)
    idx3 = token_ids.astype(jnp.int32).reshape(_NW, nchunks, _CHUNK)
    x2 = _build_gather(nchunks, total)(idx3, token_table.astype(jnp.float32))
    pos_t = jnp.tile(pos_table.astype(jnp.float32), (_RB // L, 1))
    return _ln_apply(
        x2, pos_t,
        gamma.astype(jnp.float32).reshape(1, _D),
        beta.astype(jnp.float32).reshape(1, _D),
        B, L)
